# baseline (speedup 1.0000x reference)
import functools

import jax
import jax.numpy as jnp
from jax import lax
from jax.experimental import pallas as pl
from jax.experimental.pallas import tpu as pltpu
from jax.experimental.pallas import tpu_sc as plsc

NUM_NUM = 13
NUM_CAT = 26
BATCH = 4096
VOCAB = 100000
D = 64

L = 16
CHUNK = 128
CAT_ROWS = NUM_CAT * BATCH
NUM_ROWS = NUM_NUM * BATCH
NW = 32
CAT_J = CAT_ROWS // CHUNK // NW
NUM_J = NUM_ROWS // CHUNK // NW
NBUF = 13


def _make_kernel():
    mesh = plsc.VectorSubcoreMesh(core_axis_name="c", subcore_axis_name="s")

    @functools.partial(
        pl.kernel,
        mesh=mesh,
        out_type=jax.ShapeDtypeStruct((BATCH * (NUM_CAT + NUM_NUM), D),
                                      jnp.float32),
        compiler_params=pltpu.CompilerParams(use_tc_tiling_on_sc=False),
        scratch_types=[
            pltpu.VMEM((CAT_J, CHUNK), jnp.int32),
            pltpu.VMEM((CAT_J, CHUNK), jnp.int32),
            pltpu.VMEM((NUM_J, CHUNK), jnp.int32),
            pltpu.VMEM((NUM_J, CHUNK), jnp.float32),
            pltpu.VMEM((NUM_NUM, D), jnp.float32),
            pltpu.VMEM((NBUF, CHUNK, D), jnp.float32),
            pltpu.SemaphoreType.DMA,
            pltpu.SemaphoreType.DMA,
        ],
    )
    def k(tab, idx, xnum, wnum, out, gidx, oidx, oidxn, xv, wv, bufs,
          gsem, ssem):
        nc = 2
        wid = lax.axis_index("s") * nc + lax.axis_index("c")
        iota = lax.iota(jnp.int32, L)

        pltpu.sync_copy(idx.at[wid], gidx)
        pltpu.sync_copy(xnum.at[wid], xv)
        pltpu.sync_copy(wnum, wv)

        cat_base = wid * CAT_J * CHUNK
        num_base = wid * NUM_J * CHUNK

        def cat_idx_body(t, _):
            j = t // (CHUNK // L)
            g = t % (CHUNK // L)
            r = cat_base + j * CHUNK + g * L + iota
            f = r >> 12
            b = r & (BATCH - 1)
            raw = gidx[j, pl.ds(g * L, L)]
            gidx[j, pl.ds(g * L, L)] = raw + f * (VOCAB + 1)
            oidx[j, pl.ds(g * L, L)] = b * (NUM_CAT + NUM_NUM) + f
            return 0

        lax.fori_loop(0, CAT_J * (CHUNK // L), cat_idx_body, 0)

        def num_idx_body(t, _):
            j = t // (CHUNK // L)
            g = t % (CHUNK // L)
            r = num_base + j * CHUNK + g * L + iota
            f = r >> 12
            b = r & (BATCH - 1)
            oidxn[j, pl.ds(g * L, L)] = b * (NUM_CAT + NUM_NUM) + NUM_CAT + f
            return 0

        lax.fori_loop(0, NUM_J * (CHUNK // L), num_idx_body, 0)

        for half in range(2):
            gathers = []
            for i in range(NBUF):
                j = half * NBUF + i
                gathers.append(
                    pltpu.async_copy(tab.at[gidx.at[j]], bufs.at[i], gsem))
            for d in gathers:
                d.wait()
            scatters = []
            for i in range(NBUF):
                j = half * NBUF + i
                scatters.append(
                    pltpu.async_copy(bufs.at[i], out.at[oidx.at[j]], ssem))
            for d in scatters:
                d.wait()

        for c in range(NUM_J):
            f_c = (num_base + c * CHUNK) >> 12
            wrow = [wv[f_c, pl.ds(q * L, L)] for q in range(4)]

            def num_grp_body(g, _):
                xg = xv[c, pl.ds(g * L, L)]
                for l in range(L):
                    xi = xg[l]
                    for q in range(4):
                        bufs[c, g * L + l, pl.ds(q * L, L)] = xi * wrow[q]
                return 0

            lax.fori_loop(0, CHUNK // L, num_grp_body, 0)
        nscat = [pltpu.async_copy(bufs.at[c], out.at[oidxn.at[c]], ssem)
                 for c in range(NUM_J)]
        for d in nscat:
            d.wait()

    return k


def kernel(num_features, cat_features, W_num, E_cat):
    tab = E_cat.reshape(NUM_CAT * (VOCAB + 1), D)
    idx = cat_features.astype(jnp.int32).reshape(NW, CAT_J, CHUNK)
    xnum = num_features.reshape(NW, NUM_J, CHUNK)
    wnum = W_num.reshape(NUM_NUM, D)
    out = _make_kernel()(tab, idx, xnum, wnum)
    return out.reshape(BATCH, NUM_CAT + NUM_NUM, D)


# speedup vs baseline: 11.5185x; 11.5185x over previous
import functools

import jax
import jax.numpy as jnp
from jax import lax
from jax.experimental import pallas as pl
from jax.experimental.pallas import tpu as pltpu
from jax.experimental.pallas import tpu_sc as plsc

NUM_NUM = 13
NUM_CAT = 26
BATCH = 4096
VOCAB = 100000
D = 64

L = 16
NSUB = 16
VW = 6272
VLAST = 5888
VSTRIP = 15 * VW + VLAST
BSLICE = BATCH // NSUB
MAILBOX = 8 * BATCH
GROUPS = BATCH // L


def _make_kernel():
    mesh = plsc.VectorSubcoreMesh(core_axis_name="c", subcore_axis_name="s")

    @functools.partial(
        pl.kernel,
        mesh=mesh,
        out_type=jax.ShapeDtypeStruct((NUM_CAT + NUM_NUM, D, BATCH),
                                      jnp.float32),
        compiler_params=pltpu.CompilerParams(
            use_tc_tiling_on_sc=True, needs_layout_passes=False),
        scratch_types=[
            pltpu.VMEM((8, VW), jnp.float32),
            pltpu.VMEM((8, 128), jnp.float32),
            pltpu.VMEM((8, BATCH), jnp.int32),
            pltpu.VMEM((BATCH + L,), jnp.int32),
            pltpu.VMEM((BATCH + L,), jnp.int32),
            pltpu.VMEM((2, 8 * L), jnp.float32),
            pltpu.VMEM((2, 8 * L), jnp.int32),
            pltpu.VMEM((8, BSLICE), jnp.float32),
            pltpu.VMEM((NUM_NUM, BSLICE), jnp.float32),
            pltpu.VMEM((NUM_NUM, D), jnp.float32),
            pltpu.VMEM_SHARED((MAILBOX + 2 * L,), jnp.float32),
            pltpu.SemaphoreType.DMA,
            pltpu.SemaphoreType.DMA,
        ],
    )
    def k(tab, idx, xs, ws, strip, out, vband, vstrip, vjx, vlist, blist,
          cvals, coffs, vbuf, xv, wv, mbox, gsem, ssem):
        ci = lax.axis_index("c")
        sid = lax.axis_index("s")
        iota = lax.iota(jnp.int32, L)
        vbase = sid * VW
        last = sid == NSUB - 1

        pltpu.sync_copy(xs.at[:, pl.ds(sid * BSLICE, BSLICE)], xv)
        pltpu.sync_copy(ws, wv)

        def mul8(x):
            return pl.multiple_of(x * 8, 8)

        for jb in range(4):
            nj = 8 if jb < 3 else 2
            pltpu.sync_copy(idx.at[pl.ds(jb * 8, nj)], vjx.at[pl.ds(0, nj)])

            def j_body(jr, _):
                j = jb * 8 + jr
                my = (j % 2) == ci

                @pl.when(my)
                def _process_j():
                    def build(g, nacc):
                        vv = vjx[jr, pl.ds(g * L, L)]
                        inb = (vv >= vbase) & (vv < vbase + VW)
                        cnt = plsc.all_reduce_population_count(inb)[0]
                        plsc.store_compressed(
                            vlist.at[pl.ds(nacc, L)], vv - vbase, mask=inb)
                        plsc.store_compressed(
                            blist.at[pl.ds(nacc, L)], g * L + iota,
                            mask=inb)
                        return nacc + cnt

                    n = lax.fori_loop(0, GROUPS, build, 0)
                    ngrp = (n + L - 1) // L

                    def cb_body(cb, _):
                        cb8 = mul8(cb)

                        @pl.when(~last)
                        def _stage_full():
                            pltpu.sync_copy(
                                tab.at[j, pl.ds(cb8, 8), pl.ds(vbase, VW)],
                                vband)

                        @pl.when(last)
                        def _stage_last():
                            pltpu.sync_copy(
                                tab.at[j, pl.ds(cb8, 8),
                                       pl.ds(15 * VW, VLAST)],
                                vband.at[:, pl.ds(0, VLAST)])
                            pltpu.sync_copy(
                                strip.at[pl.ds(mul8(j * 8 + cb), 8)],
                                vstrip)

                        def chunk_body(g, use_strip):
                            buf = g % 2

                            @pl.when(g >= 2)
                            def _free_buf():
                                pltpu.make_async_copy(
                                    cvals.at[0], mbox.at[coffs.at[0]],
                                    ssem).wait()

                            vloc = vlist[pl.ds(g * L, L)]
                            vb = blist[pl.ds(g * L, L)]
                            inb = (g * L + iota) < n
                            vloc = jnp.where(inb, vloc, 0)
                            if use_strip:
                                vabs = vloc + vbase
                                instrip = vabs >= VSTRIP
                                vband_i = jnp.where(instrip, 0, vloc)
                                vstrip_i = jnp.where(
                                    instrip, vabs - VSTRIP, 0)
                            for c in range(8):
                                cvec = jnp.full((L,), c, jnp.int32)
                                if use_strip:
                                    val = jnp.where(
                                        instrip,
                                        plsc.load_gather(
                                            vstrip, [cvec, vstrip_i]),
                                        plsc.load_gather(
                                            vband, [cvec, vband_i]))
                                else:
                                    val = plsc.load_gather(
                                        vband, [cvec, vloc])
                                off = jnp.where(inb, c * BATCH + vb,
                                                MAILBOX + iota)
                                cvals[buf, pl.ds(c * L, L)] = val
                                coffs[buf, pl.ds(c * L, L)] = off
                            pltpu.async_copy(
                                cvals.at[buf], mbox.at[coffs.at[buf]],
                                ssem)
                            return 0

                        @pl.when(~last)
                        def _gather_plain():
                            lax.fori_loop(
                                0, ngrp,
                                lambda g, _: chunk_body(g, False), 0)

                        @pl.when(last)
                        def _gather_strip():
                            lax.fori_loop(
                                0, ngrp,
                                lambda g, _: chunk_body(g, True), 0)

                        def drain(g, _):
                            pltpu.make_async_copy(
                                cvals.at[0], mbox.at[coffs.at[0]],
                                ssem).wait()
                            return 0

                        lax.fori_loop(0, jnp.minimum(ngrp, 2), drain, 0)
                        plsc.subcore_barrier()

                        for c in range(8):
                            pltpu.sync_copy(
                                mbox.at[pl.ds(c * BATCH + sid * BSLICE,
                                              BSLICE)],
                                vbuf.at[c])
                        pltpu.sync_copy(
                            vbuf,
                            out.at[j, pl.ds(cb8, 8),
                                   pl.ds(sid * BSLICE, BSLICE)])
                        plsc.subcore_barrier()
                        return 0

                    lax.fori_loop(0, 8, cb_body, 0)

                return 0

            lax.fori_loop(0, nj, j_body, 0)

        def num_body(t, _):
            f = t // 8
            cb = t % 8
            fmine = (f % 2) == ci

            @pl.when(fmine)
            def _num_f():
                for c in range(8):
                    col = mul8(cb) + c
                    wsplat = plsc.load_gather(
                        wv, [jnp.full((L,), f, jnp.int32),
                             jnp.full((L,), col, jnp.int32)])

                    def g_body(g, _):
                        vbuf[c, pl.ds(g * L, L)] = (
                            wsplat * xv[f, pl.ds(g * L, L)])
                        return 0

                    lax.fori_loop(0, BSLICE // L, g_body, 0)
                pltpu.sync_copy(
                    vbuf,
                    out.at[NUM_CAT + f, pl.ds(mul8(cb), 8),
                           pl.ds(sid * BSLICE, BSLICE)])
            return 0

        lax.fori_loop(0, NUM_NUM * 8, num_body, 0)

    return k


def kernel(num_features, cat_features, W_num, E_cat):
    tab = jnp.transpose(E_cat, (0, 2, 1))
    idx = cat_features.astype(jnp.int32)
    xs = num_features.reshape(NUM_NUM, BATCH)
    ws = W_num.reshape(NUM_NUM, D)
    strip = jnp.pad(tab[:, :, VSTRIP:],
                    ((0, 0), (0, 0), (0, 128 - (VOCAB + 1 - VSTRIP)))
                    ).reshape(NUM_CAT * D, 128)
    out = _make_kernel()(tab, idx, xs, ws, strip)
    return jnp.transpose(out, (2, 0, 1))


# speedup vs baseline: 17.6900x; 1.5358x over previous
import functools

import jax
import jax.numpy as jnp
from jax import lax
from jax.experimental import pallas as pl
from jax.experimental.pallas import tpu as pltpu
from jax.experimental.pallas import tpu_sc as plsc

NUM_NUM = 13
NUM_CAT = 26
BATCH = 4096
VOCAB = 100000
D = 64

L = 16
NSUB = 16
VW = 6272
VLAST = 5888
VSTRIP = 15 * VW + VLAST
BSLICE = BATCH // NSUB
MAILBOX = 8 * BATCH
GROUPS = BATCH // L


def _make_kernel():
    mesh = plsc.VectorSubcoreMesh(core_axis_name="c", subcore_axis_name="s")

    @functools.partial(
        pl.kernel,
        mesh=mesh,
        out_type=jax.ShapeDtypeStruct((NUM_CAT + NUM_NUM, D, BATCH),
                                      jnp.float32),
        compiler_params=pltpu.CompilerParams(
            use_tc_tiling_on_sc=True, needs_layout_passes=False),
        scratch_types=[
            pltpu.VMEM((2, 8, VW), jnp.float32),
            pltpu.VMEM((8, 128), jnp.float32),
            pltpu.VMEM((BATCH,), jnp.int32),
            pltpu.VMEM((BATCH + L,), jnp.int32),
            pltpu.VMEM((BATCH + L,), jnp.int32),
            pltpu.VMEM((2, 8 * L), jnp.float32),
            pltpu.VMEM((2, 8 * L), jnp.int32),
            pltpu.VMEM((8, BSLICE), jnp.float32),
            pltpu.VMEM((NUM_NUM, BSLICE), jnp.float32),
            pltpu.VMEM((NUM_NUM, D), jnp.float32),
            pltpu.VMEM_SHARED((MAILBOX + 2 * L,), jnp.float32),
            pltpu.VMEM_SHARED((8, BATCH), jnp.int32),
            pltpu.SemaphoreType.DMA,
            pltpu.SemaphoreType.DMA,
        ],
    )
    def k(tab, idx, xs, ws, strip, out, vband, vstrip, vrow, vlist, blist,
          cvals, coffs, vbuf, xv, wv, mbox, sidx, gsem, ssem):
        ci = lax.axis_index("c")
        sid = lax.axis_index("s")
        iota = lax.iota(jnp.int32, L)
        vbase = sid * VW
        last = sid == NSUB - 1

        pltpu.sync_copy(xs.at[:, pl.ds(sid * BSLICE, BSLICE)], xv)
        pltpu.sync_copy(ws, wv)

        def mul8(x):
            return pl.multiple_of(x * 8, 8)

        def stage_start(s, buf):
            t = s // 8
            j = 2 * t + ci
            cb8 = mul8(s % 8)

            @pl.when(~last)
            def _full():
                pltpu.async_copy(
                    tab.at[j, pl.ds(cb8, 8), pl.ds(vbase, VW)],
                    vband.at[buf], gsem)

            @pl.when(last)
            def _lastw():
                pltpu.async_copy(
                    tab.at[j, pl.ds(cb8, 8), pl.ds(15 * VW, VLAST)],
                    vband.at[buf, :, pl.ds(0, VLAST)], gsem)

        def stage_wait(s, buf):
            t = s // 8
            j = 2 * t + ci
            cb8 = mul8(s % 8)

            @pl.when(~last)
            def _full():
                pltpu.make_async_copy(
                    tab.at[j, pl.ds(cb8, 8), pl.ds(vbase, VW)],
                    vband.at[buf], gsem).wait()

            @pl.when(last)
            def _lastw():
                pltpu.make_async_copy(
                    tab.at[j, pl.ds(cb8, 8), pl.ds(15 * VW, VLAST)],
                    vband.at[buf, :, pl.ds(0, VLAST)], gsem).wait()

        stage_start(0, 0)

        def s_body(s, n):
            t = s // 8
            cb = s % 8
            j = 2 * t + ci
            jb = t // 4
            buf = s % 2

            @pl.when((cb == 0) & (t % 4 == 0) & (t < 12))
            def _idx_band():
                pltpu.sync_copy(idx.at[pl.ds(mul8(jb), 8)], sidx)

            @pl.when((cb == 0) & (t == 12))
            def _idx_band_tail():
                pltpu.sync_copy(idx.at[pl.ds(24, 2)], sidx.at[pl.ds(0, 2)])

            def build(_):
                pltpu.sync_copy(sidx.at[j - jb * 8], vrow)

                def build_g(g, nacc):
                    vv = vrow[pl.ds(g * L, L)]
                    inb = (vv >= vbase) & (vv < vbase + VW)
                    cnt = plsc.all_reduce_population_count(inb)[0]
                    plsc.store_compressed(
                        vlist.at[pl.ds(nacc, L)], vv - vbase, mask=inb)
                    plsc.store_compressed(
                        blist.at[pl.ds(nacc, L)], g * L + iota, mask=inb)
                    return nacc + cnt

                return lax.fori_loop(0, GROUPS, build_g, 0)

            n = lax.cond(cb == 0, build, lambda _: n, 0)
            ngrp = (n + L - 1) // L

            stage_wait(s, buf)

            @pl.when(s < 103)
            def _prefetch():
                stage_start(s + 1, 1 - buf)

            @pl.when(last)
            def _strip():
                pltpu.sync_copy(
                    strip.at[pl.ds(mul8(j * 8 + cb), 8)], vstrip)

            def chunk_body(g, use_strip):
                cbuf = g % 2

                @pl.when(g >= 2)
                def _free_buf():
                    pltpu.make_async_copy(
                        cvals.at[0], mbox.at[coffs.at[0]], ssem).wait()

                vloc = vlist[pl.ds(g * L, L)]
                vb = blist[pl.ds(g * L, L)]
                inb = (g * L + iota) < n
                vloc = jnp.where(inb, vloc, 0)
                if use_strip:
                    vabs = vloc + vbase
                    instrip = vabs >= VSTRIP
                    vband_i = jnp.where(instrip, 0, vloc)
                    vstrip_i = jnp.where(instrip, vabs - VSTRIP, 0)
                bvec = jnp.full((L,), buf, jnp.int32)
                for c in range(8):
                    cvec = jnp.full((L,), c, jnp.int32)
                    if use_strip:
                        val = jnp.where(
                            instrip,
                            plsc.load_gather(vstrip, [cvec, vstrip_i]),
                            plsc.load_gather(
                                vband, [bvec, cvec, vband_i]))
                    else:
                        val = plsc.load_gather(
                            vband, [bvec, cvec, vloc])
                    off = jnp.where(inb, c * BATCH + vb, MAILBOX + iota)
                    cvals[cbuf, pl.ds(c * L, L)] = val
                    coffs[cbuf, pl.ds(c * L, L)] = off
                pltpu.async_copy(
                    cvals.at[cbuf], mbox.at[coffs.at[cbuf]], ssem)
                return 0

            @pl.when(~last)
            def _gather_plain():
                lax.fori_loop(0, ngrp,
                              lambda g, _: chunk_body(g, False), 0)

            @pl.when(last)
            def _gather_strip():
                lax.fori_loop(0, ngrp,
                              lambda g, _: chunk_body(g, True), 0)

            def drain(g, _):
                pltpu.make_async_copy(
                    cvals.at[0], mbox.at[coffs.at[0]], ssem).wait()
                return 0

            lax.fori_loop(0, jnp.minimum(ngrp, 2), drain, 0)
            plsc.subcore_barrier()

            for c in range(8):
                pltpu.sync_copy(
                    mbox.at[pl.ds(c * BATCH + sid * BSLICE, BSLICE)],
                    vbuf.at[c])
            pltpu.sync_copy(
                vbuf,
                out.at[j, pl.ds(mul8(cb), 8),
                       pl.ds(sid * BSLICE, BSLICE)])
            plsc.subcore_barrier()
            return n

        lax.fori_loop(0, 104, s_body, 0)

        def num_body(t, _):
            f = t // 8
            cb = t % 8
            fmine = (f % 2) == ci

            @pl.when(fmine)
            def _num_f():
                for c in range(8):
                    col = mul8(cb) + c
                    wsplat = plsc.load_gather(
                        wv, [jnp.full((L,), f, jnp.int32),
                             jnp.full((L,), col, jnp.int32)])

                    def g_body(g, _):
                        vbuf[c, pl.ds(g * L, L)] = (
                            wsplat * xv[f, pl.ds(g * L, L)])
                        return 0

                    lax.fori_loop(0, BSLICE // L, g_body, 0)
                pltpu.sync_copy(
                    vbuf,
                    out.at[NUM_CAT + f, pl.ds(mul8(cb), 8),
                           pl.ds(sid * BSLICE, BSLICE)])
            return 0

        lax.fori_loop(0, NUM_NUM * 8, num_body, 0)

    return k


def kernel(num_features, cat_features, W_num, E_cat):
    tab = jnp.transpose(E_cat, (0, 2, 1))
    idx = cat_features.astype(jnp.int32)
    xs = num_features.reshape(NUM_NUM, BATCH)
    ws = W_num.reshape(NUM_NUM, D)
    strip = jnp.pad(tab[:, :, VSTRIP:],
                    ((0, 0), (0, 0), (0, 128 - (VOCAB + 1 - VSTRIP)))
                    ).reshape(NUM_CAT * D, 128)
    out = _make_kernel()(tab, idx, xs, ws, strip)
    return jnp.transpose(out, (2, 0, 1))


# speedup vs baseline: 17.9339x; 1.0138x over previous
import functools

import jax
import jax.numpy as jnp
from jax import lax
from jax.experimental import pallas as pl
from jax.experimental.pallas import tpu as pltpu
from jax.experimental.pallas import tpu_sc as plsc

NUM_NUM = 13
NUM_CAT = 26
BATCH = 4096
VOCAB = 100000
D = 64

L = 16
NSUB = 16
VW = 6272
VLAST = 5888
VSTRIP = 15 * VW + VLAST
BSLICE = BATCH // NSUB
MAILBOX = 8 * BATCH
GROUPS = BATCH // L


def _make_kernel():
    mesh = plsc.VectorSubcoreMesh(core_axis_name="c", subcore_axis_name="s")

    @functools.partial(
        pl.kernel,
        mesh=mesh,
        out_type=jax.ShapeDtypeStruct((NUM_CAT + NUM_NUM, D, BATCH),
                                      jnp.float32),
        compiler_params=pltpu.CompilerParams(
            use_tc_tiling_on_sc=True, needs_layout_passes=False),
        scratch_types=[
            pltpu.VMEM((2, 8, VW), jnp.float32),
            pltpu.VMEM((8, 128), jnp.float32),
            pltpu.VMEM((BATCH,), jnp.int32),
            pltpu.VMEM((BATCH + L,), jnp.int32),
            pltpu.VMEM((BATCH + L,), jnp.int32),
            pltpu.VMEM((2, 8 * L), jnp.float32),
            pltpu.VMEM((2, 8 * L), jnp.int32),
            pltpu.VMEM((8, BSLICE), jnp.float32),
            pltpu.VMEM((8 * BSLICE,), jnp.float32),
            pltpu.VMEM((NUM_NUM, BSLICE), jnp.float32),
            pltpu.VMEM((NUM_NUM, D), jnp.float32),
            pltpu.VMEM_SHARED((2 * MAILBOX + 2 * L,), jnp.float32),
            pltpu.VMEM_SHARED((8, BATCH), jnp.int32),
            pltpu.SemaphoreType.DMA,
            pltpu.SemaphoreType.DMA,
        ],
    )
    def k(tab, idx, xs, ws, strip, out, vband, vstrip, vrow, vlist, blist,
          cvals, coffs, vbuf, vtmp, xv, wv, mbox, sidx, gsem, ssem):
        ci = lax.axis_index("c")
        sid = lax.axis_index("s")
        iota = lax.iota(jnp.int32, L)
        vbase = sid * VW
        last = sid == NSUB - 1

        pltpu.sync_copy(xs.at[:, pl.ds(sid * BSLICE, BSLICE)], xv)
        pltpu.sync_copy(ws, wv)

        def mul8(x):
            return pl.multiple_of(x * 8, 8)

        def stage_start(s, buf):
            t = s // 8
            j = 2 * t + ci
            cb8 = mul8(s % 8)

            @pl.when(~last)
            def _full():
                pltpu.async_copy(
                    tab.at[j, pl.ds(cb8, 8), pl.ds(vbase, VW)],
                    vband.at[buf], gsem)

            @pl.when(last)
            def _lastw():
                pltpu.async_copy(
                    tab.at[j, pl.ds(cb8, 8), pl.ds(15 * VW, VLAST)],
                    vband.at[buf, :, pl.ds(0, VLAST)], gsem)

        def stage_wait(s, buf):
            t = s // 8
            j = 2 * t + ci
            cb8 = mul8(s % 8)

            @pl.when(~last)
            def _full():
                pltpu.make_async_copy(
                    tab.at[j, pl.ds(cb8, 8), pl.ds(vbase, VW)],
                    vband.at[buf], gsem).wait()

            @pl.when(last)
            def _lastw():
                pltpu.make_async_copy(
                    tab.at[j, pl.ds(cb8, 8), pl.ds(15 * VW, VLAST)],
                    vband.at[buf, :, pl.ds(0, VLAST)], gsem).wait()

        stage_start(0, 0)

        def s_body(s, n):
            t = s // 8
            cb = s % 8
            j = 2 * t + ci
            jb = t // 4
            buf = s % 2

            @pl.when((cb == 0) & (t % 4 == 0) & (t < 12))
            def _idx_band():
                pltpu.sync_copy(idx.at[pl.ds(mul8(jb), 8)], sidx)

            @pl.when((cb == 0) & (t == 12))
            def _idx_band_tail():
                pltpu.sync_copy(idx.at[pl.ds(24, 2)], sidx.at[pl.ds(0, 2)])

            def build(_):
                pltpu.sync_copy(sidx.at[j - jb * 8], vrow)

                def build_g(g, nacc):
                    vv = vrow[pl.ds(g * L, L)]
                    inb = (vv >= vbase) & (vv < vbase + VW)
                    cnt = plsc.all_reduce_population_count(inb)[0]
                    plsc.store_compressed(
                        vlist.at[pl.ds(nacc, L)], vv - vbase, mask=inb)
                    plsc.store_compressed(
                        blist.at[pl.ds(nacc, L)], g * L + iota, mask=inb)
                    return nacc + cnt

                return lax.fori_loop(0, GROUPS, build_g, 0)

            n = lax.cond(cb == 0, build, lambda _: n, 0)
            ngrp = (n + L - 1) // L

            stage_wait(s, buf)

            @pl.when(s < 103)
            def _prefetch():
                stage_start(s + 1, 1 - buf)

            @pl.when(last)
            def _strip():
                pltpu.sync_copy(
                    strip.at[pl.ds(mul8(j * 8 + cb), 8)], vstrip)

            def chunk_body(g, use_strip):
                cbuf = g % 2

                @pl.when(g >= 2)
                def _free_buf():
                    pltpu.make_async_copy(
                        cvals.at[0], mbox.at[coffs.at[0]], ssem).wait()

                vloc = vlist[pl.ds(g * L, L)]
                vb = blist[pl.ds(g * L, L)]
                inb = (g * L + iota) < n
                vloc = jnp.where(inb, vloc, 0)
                if use_strip:
                    vabs = vloc + vbase
                    instrip = vabs >= VSTRIP
                    vband_i = jnp.where(instrip, 0, vloc)
                    vstrip_i = jnp.where(instrip, vabs - VSTRIP, 0)
                bvec = jnp.full((L,), buf, jnp.int32)
                for c in range(8):
                    cvec = jnp.full((L,), c, jnp.int32)
                    if use_strip:
                        val = jnp.where(
                            instrip,
                            plsc.load_gather(vstrip, [cvec, vstrip_i]),
                            plsc.load_gather(
                                vband, [bvec, cvec, vband_i]))
                    else:
                        val = plsc.load_gather(
                            vband, [bvec, cvec, vloc])
                    off = jnp.where(inb, buf * MAILBOX + vb * 8 + c,
                                    2 * MAILBOX + iota)
                    cvals[cbuf, pl.ds(c * L, L)] = val
                    coffs[cbuf, pl.ds(c * L, L)] = off
                pltpu.async_copy(
                    cvals.at[cbuf], mbox.at[coffs.at[cbuf]], ssem)
                return 0

            @pl.when(~last)
            def _gather_plain():
                lax.fori_loop(0, ngrp,
                              lambda g, _: chunk_body(g, False), 0)

            @pl.when(last)
            def _gather_strip():
                lax.fori_loop(0, ngrp,
                              lambda g, _: chunk_body(g, True), 0)

            def drain(g, _):
                pltpu.make_async_copy(
                    cvals.at[0], mbox.at[coffs.at[0]], ssem).wait()
                return 0

            lax.fori_loop(0, jnp.minimum(ngrp, 2), drain, 0)
            plsc.subcore_barrier()

            pltpu.sync_copy(
                mbox.at[pl.ds(buf * MAILBOX + sid * BSLICE * 8,
                              BSLICE * 8)],
                vtmp)

            def tr_body(g, _):
                rows = (g * L + iota) * 8
                for c in range(8):
                    vbuf[c, pl.ds(g * L, L)] = plsc.load_gather(
                        vtmp, [rows + c])
                return 0

            lax.fori_loop(0, BSLICE // L, tr_body, 0)
            pltpu.sync_copy(
                vbuf,
                out.at[j, pl.ds(mul8(cb), 8),
                       pl.ds(sid * BSLICE, BSLICE)])
            return n

        lax.fori_loop(0, 104, s_body, 0)

        def num_body(t, _):
            f = t // 8
            cb = t % 8
            fmine = (f % 2) == ci

            @pl.when(fmine)
            def _num_f():
                for c in range(8):
                    col = mul8(cb) + c
                    wsplat = plsc.load_gather(
                        wv, [jnp.full((L,), f, jnp.int32),
                             jnp.full((L,), col, jnp.int32)])

                    def g_body(g, _):
                        vbuf[c, pl.ds(g * L, L)] = (
                            wsplat * xv[f, pl.ds(g * L, L)])
                        return 0

                    lax.fori_loop(0, BSLICE // L, g_body, 0)
                pltpu.sync_copy(
                    vbuf,
                    out.at[NUM_CAT + f, pl.ds(mul8(cb), 8),
                           pl.ds(sid * BSLICE, BSLICE)])
            return 0

        lax.fori_loop(0, NUM_NUM * 8, num_body, 0)

    return k


def kernel(num_features, cat_features, W_num, E_cat):
    tab = jnp.transpose(E_cat, (0, 2, 1))
    idx = cat_features.astype(jnp.int32)
    xs = num_features.reshape(NUM_NUM, BATCH)
    ws = W_num.reshape(NUM_NUM, D)
    strip = jnp.pad(tab[:, :, VSTRIP:],
                    ((0, 0), (0, 0), (0, 128 - (VOCAB + 1 - VSTRIP)))
                    ).reshape(NUM_CAT * D, 128)
    out = _make_kernel()(tab, idx, xs, ws, strip)
    return jnp.transpose(out, (2, 0, 1))
